# baseline (speedup 1.0000x reference)
import jax
import jax.numpy as jnp
from jax.experimental import pallas as pl
from jax.experimental.pallas import tpu as pltpu

B = 256
N = 65536
K = 10
BB = 32
G = B // BB


def _body(x_ref, rt_ref, o_ref, assoc_ref, occ_ref):
    i = pl.program_id(0)

    @pl.when(i == 0)
    def _prep():
        r = rt_ref[...]
        denom = jnp.maximum(jnp.sum(jnp.abs(r), axis=0, keepdims=True), 1e-12)
        p = r / denom
        m = jnp.max(p, axis=0, keepdims=True)
        row = jax.lax.broadcasted_iota(jnp.int32, p.shape, 0)
        ismax = p == m
        amax = jnp.min(jnp.where(ismax, row, K), axis=0, keepdims=True)
        onehot = row == amax
        assoc_ref[...] = jnp.where(onehot, p, 0.0).astype(jnp.bfloat16)
        occ_ref[...] = jnp.sum(onehot.astype(jnp.float32), axis=1)[None, :]

    x = x_ref[...].astype(jnp.bfloat16)
    y = jax.lax.dot_general(
        x, assoc_ref[...],
        dimension_numbers=(((1,), (1,)), ((), ())),
        preferred_element_type=jnp.float32,
    )
    occ = occ_ref[...]
    o_ref[...] = jnp.where(occ > 0.0, y / occ, 0.0)


@jax.jit
def kernel(inputs, rates):
    rates_t = rates.T
    out = pl.pallas_call(
        _body,
        grid=(G,),
        in_specs=[
            pl.BlockSpec((BB, N), lambda i: (i, 0)),
            pl.BlockSpec((K, N), lambda i: (0, 0)),
        ],
        out_specs=pl.BlockSpec((BB, K), lambda i: (i, 0)),
        out_shape=jax.ShapeDtypeStruct((B, K), jnp.float32),
        scratch_shapes=[
            pltpu.VMEM((K, N), jnp.bfloat16),
            pltpu.VMEM((1, K), jnp.float32),
        ],
        compiler_params=pltpu.CompilerParams(
            dimension_semantics=("arbitrary",),
        ),
    )(inputs, rates_t)
    return out


# speedup vs baseline: 1.1518x; 1.1518x over previous
import jax
import jax.numpy as jnp
from jax.experimental import pallas as pl
from jax.experimental.pallas import tpu as pltpu

B = 256
N = 65536
K = 10
BN = 8192
G = N // BN
BH = B // 2


def _body(x1_ref, x2_ref, rt_ref, o_ref, occ_ref):
    i = pl.program_id(0)

    @pl.when(i == 0)
    def _init():
        o_ref[...] = jnp.zeros_like(o_ref)
        occ_ref[...] = jnp.zeros_like(occ_ref)

    r = rt_ref[...]
    denom = jnp.maximum(jnp.sum(jnp.abs(r), axis=0, keepdims=True), 1e-12)
    p = r / denom
    m = jnp.max(p, axis=0, keepdims=True)
    row = jax.lax.broadcasted_iota(jnp.int32, p.shape, 0)
    ismax = p == m
    amax = jnp.min(jnp.where(ismax, row, K), axis=0, keepdims=True)
    onehot = row == amax
    assoc = jnp.where(onehot, p, 0.0).astype(jnp.bfloat16)

    dn = (((1,), (1,)), ((), ()))
    x1 = x1_ref[...].astype(jnp.bfloat16)
    o_ref[0:BH, :] += jax.lax.dot_general(
        x1, assoc, dimension_numbers=dn, preferred_element_type=jnp.float32)
    x2 = x2_ref[...].astype(jnp.bfloat16)
    o_ref[BH:B, :] += jax.lax.dot_general(
        x2, assoc, dimension_numbers=dn, preferred_element_type=jnp.float32)
    occ_ref[...] += jnp.sum(onehot.astype(jnp.float32), axis=1)[None, :]

    @pl.when(i == G - 1)
    def _finish():
        occ = occ_ref[...]
        y = o_ref[...]
        o_ref[...] = jnp.where(occ > 0.0, y / occ, 0.0)


@jax.jit
def kernel(inputs, rates):
    rates_t = rates.T
    out = pl.pallas_call(
        _body,
        grid=(G,),
        in_specs=[
            pl.BlockSpec((BH, BN), lambda i: (0, i)),
            pl.BlockSpec((BH, BN), lambda i: (1, i)),
            pl.BlockSpec((K, BN), lambda i: (0, i)),
        ],
        out_specs=pl.BlockSpec((B, K), lambda i: (0, 0)),
        out_shape=jax.ShapeDtypeStruct((B, K), jnp.float32),
        scratch_shapes=[pltpu.VMEM((1, K), jnp.float32)],
        compiler_params=pltpu.CompilerParams(
            dimension_semantics=("arbitrary",),
        ),
    )(inputs, inputs, rates_t)
    return out
